# baseline (speedup 1.0000x reference)
import jax
import jax.numpy as jnp
from jax import lax
from jax.experimental import pallas as pl
from jax.experimental.pallas import tpu as pltpu
from jax.experimental.pallas import tpu_sc as plsc

N_NODES = 10000
N_EDGES = 320000
D = 128
DH = D // 2
OUT = 40

NC, NS = 2, 16
E_TILE = N_EDGES // NS
CH = 80
NCH = E_TILE // CH
NBUF = 5
DEGW = 16

_mesh = plsc.VectorSubcoreMesh(core_axis_name="c", subcore_axis_name="s")


def _sc_agg_body(h_hbm, src_hbm, dst_hbm, zagg_hbm, agg_out,
                 src_v, dst_v, rows_v, agg_sh, *sems):
    c = lax.axis_index("c")
    s = lax.axis_index("s")

    @pl.when(s == 0)
    def _():
        pltpu.sync_copy(zagg_hbm, agg_sh)

    pltpu.sync_copy(src_hbm.at[s], src_v)
    pltpu.sync_copy(dst_hbm.at[s], dst_v)
    plsc.subcore_barrier()

    h_c = h_hbm.at[c]
    gsem = sems

    for b in range(NBUF - 1):
        pltpu.async_copy(h_c.at[src_v.at[b]], rows_v.at[b], gsem[b])

    def group(i, carry):
        for k in range(NBUF):
            j = NBUF * i + k
            pltpu.make_async_copy(h_c.at[src_v.at[j]], rows_v.at[k],
                                  gsem[k]).wait()
            kn = (k + NBUF - 1) % NBUF

            @pl.when(j + NBUF - 1 < NCH)
            def _():
                pltpu.async_copy(h_c.at[src_v.at[j + NBUF - 1]],
                                 rows_v.at[kn], gsem[kn])

            pltpu.sync_copy(rows_v.at[k], agg_sh.at[dst_v.at[j]], add=True)
        return carry

    lax.fori_loop(0, NCH // NBUF, group, None)

    plsc.subcore_barrier()

    @pl.when(s == 0)
    def _():
        pltpu.sync_copy(agg_sh, agg_out.at[c])


_sc_agg = pl.kernel(
    _sc_agg_body,
    out_type=(jax.ShapeDtypeStruct((NC, N_NODES, DH), jnp.float32),),
    mesh=_mesh,
    compiler_params=pltpu.CompilerParams(use_tc_tiling_on_sc=False),
    scratch_types=[
        pltpu.VMEM((NCH, CH), jnp.int32),
        pltpu.VMEM((NCH, CH), jnp.int32),
        pltpu.VMEM((NBUF, CH, DH), jnp.float32),
        pltpu.VMEM_SHARED((N_NODES, DH), jnp.float32),
    ] + [pltpu.SemaphoreType.DMA] * NBUF,
)


def _sc_deg_body(dst_hbm, zdeg_hbm, ones_hbm, deg_out,
                 dst_v, ones_v, deg_sh):
    c = lax.axis_index("c")
    s = lax.axis_index("s")

    @pl.when(s == 0)
    def _():
        pltpu.sync_copy(zdeg_hbm, deg_sh)

    pltpu.sync_copy(ones_hbm, ones_v)
    pltpu.sync_copy(dst_hbm.at[s], dst_v)
    plsc.subcore_barrier()

    half = NCH // 2

    def step(j, carry):
        pltpu.sync_copy(ones_v, deg_sh.at[dst_v.at[c * half + j]], add=True)
        return carry

    lax.fori_loop(0, half, step, None)
    plsc.subcore_barrier()

    @pl.when(s == 0)
    def _():
        pltpu.sync_copy(deg_sh, deg_out.at[c])


_sc_deg = pl.kernel(
    _sc_deg_body,
    out_type=(jax.ShapeDtypeStruct((NC, N_NODES, DEGW), jnp.float32),),
    mesh=_mesh,
    compiler_params=pltpu.CompilerParams(use_tc_tiling_on_sc=False),
    scratch_types=[
        pltpu.VMEM((NCH, CH), jnp.int32),
        pltpu.VMEM((CH, DEGW), jnp.float32),
        pltpu.VMEM_SHARED((N_NODES, DEGW), jnp.float32),
    ],
)


RB = 10000


def _tc_layer_body(aggp_ref, degp_ref, w_ref, b_ref, out_ref):
    agg = jnp.concatenate([aggp_ref[0], aggp_ref[1]], axis=1)
    deg = degp_ref[0] + degp_ref[1]
    deg0 = jnp.maximum(deg[:, 0:1], 1.0)
    h = agg / deg0
    acc = jnp.dot(h, w_ref[...], preferred_element_type=jnp.float32)
    h = jnp.maximum(acc + b_ref[...], 0.0)
    out_ref[0] = h[:, :DH]
    out_ref[1] = h[:, DH:]


def _tc_layer(aggp, degp, w, b):
    grid = N_NODES // RB
    return pl.pallas_call(
        _tc_layer_body,
        grid=(grid,),
        in_specs=[
            pl.BlockSpec((NC, RB, DH), lambda i: (0, i, 0)),
            pl.BlockSpec((NC, RB, DEGW), lambda i: (0, i, 0)),
            pl.BlockSpec((D, D), lambda i: (0, 0)),
            pl.BlockSpec((1, D), lambda i: (0, 0)),
        ],
        out_specs=pl.BlockSpec((NC, RB, DH), lambda i: (0, i, 0)),
        out_shape=jax.ShapeDtypeStruct((NC, N_NODES, DH), jnp.float32),
    )(aggp, degp, w, b)


def _tc_final_body(aggp_ref, degp_ref, w3_ref, b3_ref, wf1_ref, bf1_ref,
                   wf2_ref, bf2_ref, out_ref):
    agg = jnp.concatenate([aggp_ref[0], aggp_ref[1]], axis=1)
    deg = degp_ref[0] + degp_ref[1]
    deg0 = jnp.maximum(deg[:, 0:1], 1.0)
    h = agg / deg0
    h = jnp.maximum(
        jnp.dot(h, w3_ref[...], preferred_element_type=jnp.float32)
        + b3_ref[...], 0.0)
    h = jnp.maximum(
        jnp.dot(h, wf1_ref[...], preferred_element_type=jnp.float32)
        + bf1_ref[...], 0.0)
    logits = (jnp.dot(h, wf2_ref[...], preferred_element_type=jnp.float32)
              + bf2_ref[...])
    col = lax.broadcasted_iota(jnp.int32, logits.shape, 1)
    logits = jnp.where(col < OUT, logits, -1e30)
    m = jnp.max(logits, axis=1, keepdims=True)
    e = jnp.exp(logits - m)
    lse = jnp.log(jnp.sum(e, axis=1, keepdims=True))
    out_ref[...] = logits - m - lse


def _tc_final(aggp, degp, w3, b3, wf1, bf1, wf2p, bf2p):
    grid = N_NODES // RB
    full = lambda r, c_: pl.BlockSpec((r, c_), lambda i: (0, 0))
    return pl.pallas_call(
        _tc_final_body,
        grid=(grid,),
        in_specs=[
            pl.BlockSpec((NC, RB, DH), lambda i: (0, i, 0)),
            pl.BlockSpec((NC, RB, DEGW), lambda i: (0, i, 0)),
            full(D, D), full(1, D),
            full(D, D), full(1, D),
            full(D, D), full(1, D),
        ],
        out_specs=pl.BlockSpec((RB, D), lambda i: (i, 0)),
        out_shape=jax.ShapeDtypeStruct((N_NODES, D), jnp.float32),
    )(aggp, degp, w3, b3, wf1, bf1, wf2p, bf2p)


def kernel(x, edge_index, W1, b1, W2, b2, W3, b3, Wf1, bf1, Wf2, bf2):
    ei = edge_index.astype(jnp.int32)
    src = ei[0].reshape(NS, NCH, CH)
    dst = ei[1].reshape(NS, NCH, CH)
    zagg = jnp.zeros((N_NODES, DH), jnp.float32)
    zdeg = jnp.zeros((N_NODES, DEGW), jnp.float32)
    ones = jnp.ones((CH, DEGW), jnp.float32)
    x2 = jnp.stack([x[:, :DH], x[:, DH:]])

    (degp,) = _sc_deg(dst, zdeg, ones)
    zagg, degp = jax.lax.optimization_barrier((zagg, degp))
    (aggp,) = _sc_agg(x2, src, dst, zagg)
    h2 = _tc_layer(aggp, degp, W1, b1.reshape(1, D))
    (aggp,) = _sc_agg(h2, src, dst, zagg)
    h2 = _tc_layer(aggp, degp, W2, b2.reshape(1, D))
    (aggp,) = _sc_agg(h2, src, dst, zagg)

    wf2p = jnp.zeros((D, D), jnp.float32).at[:, :OUT].set(Wf2)
    bf2p = jnp.zeros((1, D), jnp.float32).at[0, :OUT].set(bf2)
    out = _tc_final(aggp, degp, W3, b3.reshape(1, D),
                    Wf1, bf1.reshape(1, D), wf2p, bf2p)
    return out[:, :OUT]


# speedup vs baseline: 1.0627x; 1.0627x over previous
import jax
import jax.numpy as jnp
from jax import lax
from jax.experimental import pallas as pl
from jax.experimental.pallas import tpu as pltpu
from jax.experimental.pallas import tpu_sc as plsc

N_NODES = 10000
N_EDGES = 320000
D = 128
DH = D // 2
OUT = 40
NP = N_NODES // 2

NC, NS = 2, 16
E_TILE = N_EDGES // NS
CH = 80
NCH = E_TILE // CH
NBUF = 5

_mesh = plsc.VectorSubcoreMesh(core_axis_name="c", subcore_axis_name="s")


def _sc_agg_body(h_hbm, src_hbm, dst_hbm, zagg_hbm, agg_out,
                 src_v, dst_v, rows_v, agg_sh, *sems):
    c = lax.axis_index("c")
    s = lax.axis_index("s")

    @pl.when(s == 0)
    def _():
        pltpu.sync_copy(zagg_hbm, agg_sh)

    pltpu.sync_copy(src_hbm.at[s], src_v)
    pltpu.sync_copy(dst_hbm.at[s], dst_v)
    plsc.subcore_barrier()

    h_c = h_hbm.at[c]
    gsem = sems

    for b in range(NBUF - 1):
        pltpu.async_copy(h_c.at[src_v.at[b]], rows_v.at[b], gsem[b])

    def group(i, carry):
        for k in range(NBUF):
            j = NBUF * i + k
            pltpu.make_async_copy(h_c.at[src_v.at[j]], rows_v.at[k],
                                  gsem[k]).wait()
            kn = (k + NBUF - 1) % NBUF

            @pl.when(j + NBUF - 1 < NCH)
            def _():
                pltpu.async_copy(h_c.at[src_v.at[j + NBUF - 1]],
                                 rows_v.at[kn], gsem[kn])

            pltpu.sync_copy(rows_v.at[k], agg_sh.at[dst_v.at[j]], add=True)
        return carry

    lax.fori_loop(0, NCH // NBUF, group, None)

    plsc.subcore_barrier()

    @pl.when(s == 0)
    def _():
        pltpu.sync_copy(agg_sh, agg_out.at[c])


_sc_agg = pl.kernel(
    _sc_agg_body,
    out_type=(jax.ShapeDtypeStruct((NC, N_NODES, DH), jnp.float32),),
    mesh=_mesh,
    compiler_params=pltpu.CompilerParams(use_tc_tiling_on_sc=False),
    scratch_types=[
        pltpu.VMEM((NCH, CH), jnp.int32),
        pltpu.VMEM((NCH, CH), jnp.int32),
        pltpu.VMEM((NBUF, CH, DH), jnp.float32),
        pltpu.VMEM_SHARED((N_NODES, DH), jnp.float32),
    ] + [pltpu.SemaphoreType.DMA] * NBUF,
)


HB = 4000
HG = N_EDGES // HB


def _tc_hist_body(dst_ref, c_ref):
    i = pl.program_id(0)

    @pl.when(i == 0)
    def _():
        c_ref[...] = jnp.zeros_like(c_ref)

    v = dst_ref[0]
    lane = lax.broadcasted_iota(jnp.int32, (D, HB), 0)
    e_hi = (v // 100 == lane).astype(jnp.bfloat16)
    e_lo = (v % 100 == lane).astype(jnp.bfloat16)
    c_ref[...] += lax.dot_general(
        e_hi, e_lo, (((1,), (1,)), ((), ())),
        preferred_element_type=jnp.float32)


def _tc_hist(dst3):
    return pl.pallas_call(
        _tc_hist_body,
        grid=(HG,),
        in_specs=[pl.BlockSpec((1, 1, HB), lambda i: (i, 0, 0))],
        out_specs=pl.BlockSpec((D, D), lambda i: (0, 0)),
        out_shape=jax.ShapeDtypeStruct((D, D), jnp.float32),
    )(dst3)


RB = 1000


def _packed_recip_deg(c_ref, m0, rows):
    mi = lax.broadcasted_iota(jnp.int32, (rows, D), 0) + m0
    lane = lax.broadcasted_iota(jnp.int32, (rows, D), 1)
    cols = []
    for par in (0, 1):
        n = 2 * mi + par
        e_hi = (n // 100 == lane).astype(jnp.float32)
        e_lo = (n % 100 == lane).astype(jnp.float32)
        rowsum = jnp.dot(e_hi, c_ref[...],
                         preferred_element_type=jnp.float32)
        deg = jnp.sum(rowsum * e_lo, axis=1, keepdims=True)
        cols.append(1.0 / jnp.maximum(deg, 1.0))
    return jnp.concatenate(
        [jnp.broadcast_to(cols[0], (rows, DH)),
         jnp.broadcast_to(cols[1], (rows, DH))], axis=1)


def _tc_recip_body(c_ref, r_ref):
    i = pl.program_id(0)
    r_ref[...] = _packed_recip_deg(c_ref, i * RB, RB)


def _tc_recip(cmat):
    return pl.pallas_call(
        _tc_recip_body,
        grid=(NP // RB,),
        in_specs=[pl.BlockSpec((D, D), lambda i: (0, 0))],
        out_specs=pl.BlockSpec((RB, D), lambda i: (i, 0)),
        out_shape=jax.ShapeDtypeStruct((NP, D), jnp.float32),
    )(cmat)


def _unpack_norm(p_ref, r_ref):
    p0 = p_ref[0]
    p1 = p_ref[1]
    r = r_ref[...]
    a_e = jnp.concatenate([p0[:, :DH], p1[:, :DH]], axis=1)
    a_o = jnp.concatenate([p0[:, DH:], p1[:, DH:]], axis=1)
    h_e = a_e * r[:, 0:1]
    h_o = a_o * r[:, DH:DH + 1]
    return h_e, h_o


def _tc_layer_body(p_ref, r_ref, w_ref, b_ref, out_ref):
    h_e, h_o = _unpack_norm(p_ref, r_ref)
    w = w_ref[...]
    b = b_ref[...]
    h_e = jnp.maximum(jnp.dot(h_e, w, preferred_element_type=jnp.float32)
                      + b, 0.0)
    h_o = jnp.maximum(jnp.dot(h_o, w, preferred_element_type=jnp.float32)
                      + b, 0.0)
    out_ref[0] = jnp.concatenate([h_e[:, :DH], h_o[:, :DH]], axis=1)
    out_ref[1] = jnp.concatenate([h_e[:, DH:], h_o[:, DH:]], axis=1)


def _tc_layer(pp, rp, w, b):
    return pl.pallas_call(
        _tc_layer_body,
        grid=(NP // RB,),
        in_specs=[
            pl.BlockSpec((NC, RB, D), lambda i: (0, i, 0)),
            pl.BlockSpec((RB, D), lambda i: (i, 0)),
            pl.BlockSpec((D, D), lambda i: (0, 0)),
            pl.BlockSpec((1, D), lambda i: (0, 0)),
        ],
        out_specs=pl.BlockSpec((NC, RB, D), lambda i: (0, i, 0)),
        out_shape=jax.ShapeDtypeStruct((NC, NP, D), jnp.float32),
    )(pp, rp, w, b)


def _head(h, w3, b3, wf1, bf1, wf2, bf2):
    h = jnp.maximum(jnp.dot(h, w3, preferred_element_type=jnp.float32)
                    + b3, 0.0)
    h = jnp.maximum(jnp.dot(h, wf1, preferred_element_type=jnp.float32)
                    + bf1, 0.0)
    logits = jnp.dot(h, wf2, preferred_element_type=jnp.float32) + bf2
    col = lax.broadcasted_iota(jnp.int32, logits.shape, 1)
    logits = jnp.where(col < OUT, logits, -1e30)
    m = jnp.max(logits, axis=1, keepdims=True)
    e = jnp.exp(logits - m)
    lse = jnp.log(jnp.sum(e, axis=1, keepdims=True))
    return logits - m - lse


def _tc_final_body(p_ref, r_ref, w3_ref, b3_ref, wf1_ref, bf1_ref,
                   wf2_ref, bf2_ref, out_ref):
    h_e, h_o = _unpack_norm(p_ref, r_ref)
    args = (w3_ref[...], b3_ref[...], wf1_ref[...], bf1_ref[...],
            wf2_ref[...], bf2_ref[...])
    out_ref[0] = _head(h_e, *args)
    out_ref[1] = _head(h_o, *args)


def _tc_final(pp, rp, w3, b3, wf1, bf1, wf2p, bf2p):
    full = lambda r_, c_: pl.BlockSpec((r_, c_), lambda i: (0, 0))
    return pl.pallas_call(
        _tc_final_body,
        grid=(NP // RB,),
        in_specs=[
            pl.BlockSpec((NC, RB, D), lambda i: (0, i, 0)),
            pl.BlockSpec((RB, D), lambda i: (i, 0)),
            full(D, D), full(1, D),
            full(D, D), full(1, D),
            full(D, D), full(1, D),
        ],
        out_specs=pl.BlockSpec((NC, RB, D), lambda i: (0, i, 0)),
        out_shape=jax.ShapeDtypeStruct((NC, NP, D), jnp.float32),
    )(pp, rp, w3, b3, wf1, bf1, wf2p, bf2p)


def kernel(x, edge_index, W1, b1, W2, b2, W3, b3, Wf1, bf1, Wf2, bf2):
    ei = edge_index.astype(jnp.int32)
    src = ei[0].reshape(NS, NCH, CH)
    dst = ei[1].reshape(NS, NCH, CH)
    dst3 = ei[1].reshape(HG, 1, HB)
    zagg = jnp.zeros((N_NODES, DH), jnp.float32)

    xe, xo = x[0::2], x[1::2]
    x2p = jnp.stack([
        jnp.concatenate([xe[:, :DH], xo[:, :DH]], axis=1),
        jnp.concatenate([xe[:, DH:], xo[:, DH:]], axis=1)])

    rp = _tc_recip(_tc_hist(dst3))

    (aggp,) = _sc_agg(x2p.reshape(NC, N_NODES, DH), src, dst, zagg)
    pp = _tc_layer(aggp.reshape(NC, NP, D), rp, W1, b1.reshape(1, D))
    (aggp,) = _sc_agg(pp.reshape(NC, N_NODES, DH), src, dst, zagg)
    pp = _tc_layer(aggp.reshape(NC, NP, D), rp, W2, b2.reshape(1, D))
    (aggp,) = _sc_agg(pp.reshape(NC, N_NODES, DH), src, dst, zagg)

    wf2p = jnp.zeros((D, D), jnp.float32).at[:, :OUT].set(Wf2)
    bf2p = jnp.zeros((1, D), jnp.float32).at[0, :OUT].set(bf2)
    outp = _tc_final(aggp.reshape(NC, NP, D), rp, W3, b3.reshape(1, D),
                     Wf1, bf1.reshape(1, D), wf2p, bf2p)
    return jnp.stack([outp[0, :, :OUT], outp[1, :, :OUT]],
                     axis=1).reshape(N_NODES, OUT)


# speedup vs baseline: 1.2127x; 1.1412x over previous
import jax
import jax.numpy as jnp
from jax import lax
from jax.experimental import pallas as pl
from jax.experimental.pallas import tpu as pltpu
from jax.experimental.pallas import tpu_sc as plsc

N_NODES = 10000
N_EDGES = 320000
D = 128
DH = D // 2
OUT = 40
NP = N_NODES // 2

NC, NS = 2, 16
E_TILE = N_EDGES // NS
CH = 80
NCH = E_TILE // CH
NBUF = 5

_mesh = plsc.VectorSubcoreMesh(core_axis_name="c", subcore_axis_name="s")


def _sc_agg_body(h_hbm, src_hbm, dst_hbm, zagg_hbm, agg_out,
                 src_v, dst_v, rows_v, agg_sh, *sems):
    c = lax.axis_index("c")
    s = lax.axis_index("s")

    @pl.when(s == 0)
    def _():
        pltpu.sync_copy(zagg_hbm, agg_sh)

    pltpu.sync_copy(src_hbm.at[s], src_v)
    pltpu.sync_copy(dst_hbm.at[s], dst_v)
    plsc.subcore_barrier()

    h_c = h_hbm.at[c]
    gsem = sems

    for b in range(NBUF - 1):
        pltpu.async_copy(h_c.at[src_v.at[b]], rows_v.at[b], gsem[b])

    def group(i, carry):
        for k in range(NBUF):
            j = NBUF * i + k
            pltpu.make_async_copy(h_c.at[src_v.at[j]], rows_v.at[k],
                                  gsem[k]).wait()
            kn = (k + NBUF - 1) % NBUF

            @pl.when(j + NBUF - 1 < NCH)
            def _():
                pltpu.async_copy(h_c.at[src_v.at[j + NBUF - 1]],
                                 rows_v.at[kn], gsem[kn])

            pltpu.sync_copy(rows_v.at[k], agg_sh.at[dst_v.at[j]], add=True)
        return carry

    lax.fori_loop(0, NCH // NBUF, group, None)

    plsc.subcore_barrier()

    @pl.when(s == 0)
    def _():
        pltpu.sync_copy(agg_sh, agg_out.at[c])


_sc_agg = pl.kernel(
    _sc_agg_body,
    out_type=(jax.ShapeDtypeStruct((NC, N_NODES, DH), jnp.float32),),
    mesh=_mesh,
    compiler_params=pltpu.CompilerParams(use_tc_tiling_on_sc=False),
    scratch_types=[
        pltpu.VMEM((NCH, CH), jnp.int32),
        pltpu.VMEM((NCH, CH), jnp.int32),
        pltpu.VMEM((NBUF, CH, DH), jnp.float32),
        pltpu.VMEM_SHARED((N_NODES, DH), jnp.float32),
    ] + [pltpu.SemaphoreType.DMA] * NBUF,
)


HB = 4000
HG = N_EDGES // HB


def _tc_hist_body(dst_ref, c_ref):
    i = pl.program_id(0)

    @pl.when(i == 0)
    def _():
        c_ref[...] = jnp.zeros_like(c_ref)

    v = dst_ref[0]
    lane = lax.broadcasted_iota(jnp.int32, (D, HB), 0)
    e_hi = (v // 100 == lane).astype(jnp.bfloat16)
    e_lo = (v % 100 == lane).astype(jnp.bfloat16)
    c_ref[...] += lax.dot_general(
        e_hi, e_lo, (((1,), (1,)), ((), ())),
        preferred_element_type=jnp.float32)


def _tc_hist(dst3):
    return pl.pallas_call(
        _tc_hist_body,
        grid=(HG,),
        in_specs=[pl.BlockSpec((1, 1, HB), lambda i: (i, 0, 0))],
        out_specs=pl.BlockSpec((D, D), lambda i: (0, 0)),
        out_shape=jax.ShapeDtypeStruct((D, D), jnp.float32),
    )(dst3)


RB = 1000


def _packed_recip_deg(c_ref, m0, rows):
    mi = lax.broadcasted_iota(jnp.int32, (rows, D), 0) + m0
    lane = lax.broadcasted_iota(jnp.int32, (rows, D), 1)
    cols = []
    for par in (0, 1):
        n = 2 * mi + par
        e_hi = (n // 100 == lane).astype(jnp.float32)
        e_lo = (n % 100 == lane).astype(jnp.float32)
        rowsum = jnp.dot(e_hi, c_ref[...],
                         preferred_element_type=jnp.float32)
        deg = jnp.sum(rowsum * e_lo, axis=1, keepdims=True)
        cols.append(1.0 / jnp.maximum(deg, 1.0))
    return jnp.concatenate(
        [jnp.broadcast_to(cols[0], (rows, DH)),
         jnp.broadcast_to(cols[1], (rows, DH))], axis=1)


def _tc_recip_body(c_ref, r_ref):
    i = pl.program_id(0)
    r_ref[...] = _packed_recip_deg(c_ref, i * RB, RB)


def _tc_recip(cmat):
    return pl.pallas_call(
        _tc_recip_body,
        grid=(NP // RB,),
        in_specs=[pl.BlockSpec((D, D), lambda i: (0, 0))],
        out_specs=pl.BlockSpec((RB, D), lambda i: (i, 0)),
        out_shape=jax.ShapeDtypeStruct((NP, D), jnp.float32),
    )(cmat)


def _unpack_norm(p_ref, r_ref):
    p0 = p_ref[0]
    p1 = p_ref[1]
    r = r_ref[...]
    a_e = jnp.concatenate([p0[:, :DH], p1[:, :DH]], axis=1)
    a_o = jnp.concatenate([p0[:, DH:], p1[:, DH:]], axis=1)
    h_e = a_e * r[:, 0:1]
    h_o = a_o * r[:, DH:DH + 1]
    return h_e, h_o


def _tc_layer_body(p_ref, r_ref, w_ref, b_ref, out_ref):
    h_e, h_o = _unpack_norm(p_ref, r_ref)
    w = w_ref[...]
    b = b_ref[...]
    h_e = jnp.maximum(jnp.dot(h_e, w, preferred_element_type=jnp.float32)
                      + b, 0.0)
    h_o = jnp.maximum(jnp.dot(h_o, w, preferred_element_type=jnp.float32)
                      + b, 0.0)
    out_ref[0] = jnp.concatenate([h_e[:, :DH], h_o[:, :DH]], axis=1)
    out_ref[1] = jnp.concatenate([h_e[:, DH:], h_o[:, DH:]], axis=1)


def _tc_layer(pp, rp, w, b):
    return pl.pallas_call(
        _tc_layer_body,
        grid=(NP // RB,),
        in_specs=[
            pl.BlockSpec((NC, RB, D), lambda i: (0, i, 0)),
            pl.BlockSpec((RB, D), lambda i: (i, 0)),
            pl.BlockSpec((D, D), lambda i: (0, 0)),
            pl.BlockSpec((1, D), lambda i: (0, 0)),
        ],
        out_specs=pl.BlockSpec((NC, RB, D), lambda i: (0, i, 0)),
        out_shape=jax.ShapeDtypeStruct((NC, NP, D), jnp.float32),
    )(pp, rp, w, b)


def _head(h, w3, b3, wf1, bf1, wf2, bf2):
    h = jnp.maximum(jnp.dot(h, w3, preferred_element_type=jnp.float32)
                    + b3, 0.0)
    h = jnp.maximum(jnp.dot(h, wf1, preferred_element_type=jnp.float32)
                    + bf1, 0.0)
    logits = jnp.dot(h, wf2, preferred_element_type=jnp.float32) + bf2
    col = lax.broadcasted_iota(jnp.int32, logits.shape, 1)
    logits = jnp.where(col < OUT, logits, -1e30)
    m = jnp.max(logits, axis=1, keepdims=True)
    e = jnp.exp(logits - m)
    lse = jnp.log(jnp.sum(e, axis=1, keepdims=True))
    return logits - m - lse


def _tc_final_body(p_ref, r_ref, w3_ref, b3_ref, wf1_ref, bf1_ref,
                   wf2_ref, bf2_ref, out_ref):
    h_e, h_o = _unpack_norm(p_ref, r_ref)
    args = (w3_ref[...], b3_ref[...], wf1_ref[...], bf1_ref[...],
            wf2_ref[...], bf2_ref[...])
    o_e = _head(h_e, *args)
    o_o = _head(h_o, *args)
    out_ref[...] = jnp.stack([o_e, o_o], axis=1).reshape(2 * RB, D)


def _tc_final(pp, rp, w3, b3, wf1, bf1, wf2p, bf2p):
    full = lambda r_, c_: pl.BlockSpec((r_, c_), lambda i: (0, 0))
    return pl.pallas_call(
        _tc_final_body,
        grid=(NP // RB,),
        in_specs=[
            pl.BlockSpec((NC, RB, D), lambda i: (0, i, 0)),
            pl.BlockSpec((RB, D), lambda i: (i, 0)),
            full(D, D), full(1, D),
            full(D, D), full(1, D),
            full(D, D), full(1, D),
        ],
        out_specs=pl.BlockSpec((2 * RB, D), lambda i: (i, 0)),
        out_shape=jax.ShapeDtypeStruct((N_NODES, D), jnp.float32),
    )(pp, rp, w3, b3, wf1, bf1, wf2p, bf2p)


def kernel(x, edge_index, W1, b1, W2, b2, W3, b3, Wf1, bf1, Wf2, bf2):
    ei = edge_index.astype(jnp.int32)
    src = ei[0].reshape(NS, NCH, CH)
    dst = ei[1].reshape(NS, NCH, CH)
    dst3 = ei[1].reshape(HG, 1, HB)
    zagg = jnp.zeros((N_NODES, DH), jnp.float32)

    x2 = jnp.stack([x[:, :DH], x[:, DH:]])

    rp = _tc_recip(_tc_hist(dst3))

    (aggp,) = _sc_agg(x2, src, dst, zagg)
    pp = _tc_layer(aggp.reshape(NC, NP, D), rp, W1, b1.reshape(1, D))
    (aggp,) = _sc_agg(pp.reshape(NC, N_NODES, DH), src, dst, zagg)
    pp = _tc_layer(aggp.reshape(NC, NP, D), rp, W2, b2.reshape(1, D))
    (aggp,) = _sc_agg(pp.reshape(NC, N_NODES, DH), src, dst, zagg)

    wf2p = jnp.zeros((D, D), jnp.float32).at[:, :OUT].set(Wf2)
    bf2p = jnp.zeros((1, D), jnp.float32).at[0, :OUT].set(bf2)
    out = _tc_final(aggp.reshape(NC, NP, D), rp, W3, b3.reshape(1, D),
                    Wf1, bf1.reshape(1, D), wf2p, bf2p)
    return out[:, :OUT]
